# baseline (speedup 1.0000x reference)
import functools

import jax
import jax.numpy as jnp
from jax import lax
from jax.experimental import pallas as pl
from jax.experimental.pallas import tpu as pltpu
from jax.experimental.pallas import tpu_sc as plsc

B, C, H, W = 16, 768, 48, 48
D = H * W
K = 192
NW = 32
ROWS_PER_W = B * K // NW
CHUNK = 24
NCHUNK = ROWS_PER_W // CHUNK


def _select_body(x_ref, src_ref, e_ref):
    b = pl.program_id(0)
    xb = x_ref[...]
    s = jnp.sum(xb * xb, axis=(0, 2))[None, :]

    @pl.when(b == 0)
    def _():
        e_ref[...] = s

    @pl.when(b > 0)
    def _():
        e_ref[...] = e_ref[...] + s

    @pl.when(b == B - 1)
    def _():
        erow = e_ref[...]
        row_i = lax.broadcasted_iota(jnp.int32, (C, C), 0)
        col_i = lax.broadcasted_iota(jnp.int32, (C, C), 1)
        eye = (row_i == col_i).astype(jnp.float32)

        def split3(v):
            hi = v.astype(jnp.bfloat16).astype(jnp.float32)
            r1 = v - hi
            mid = r1.astype(jnp.bfloat16).astype(jnp.float32)
            return hi, mid, r1 - mid

        def transpose_row(v):
            out = jnp.zeros((C, 1), jnp.float32)
            for p in split3(v):
                out = out + lax.dot_general(eye, p, (((1,), (1,)), ((), ())),
                                            preferred_element_type=jnp.float32)
            return out

        ecol = transpose_row(erow)
        gt = ecol > erow
        eq = ecol == erow
        jlt = row_i < col_i
        m = jnp.where(gt | (eq & jlt), 1.0, 0.0)
        ones = jnp.ones((1, C), jnp.float32)
        rank = lax.dot_general(ones, m, (((1,), (0,)), ((), ())),
                               preferred_element_type=jnp.float32)
        rank_col = transpose_row(rank)
        slot = lax.broadcasted_iota(jnp.int32, (C, K), 1).astype(jnp.float32)
        onehot = jnp.where(rank_col == slot, 1.0, 0.0)
        cids = lax.broadcasted_iota(jnp.int32, (1, C), 1).astype(jnp.float32)
        inv = jnp.zeros((1, K), jnp.float32)
        for p in split3(cids):
            inv = inv + lax.dot_general(p, onehot, (((1,), (0,)), ((), ())),
                                        preferred_element_type=jnp.float32)
        binc = lax.broadcasted_iota(jnp.int32, (B, K), 0) * C
        src_ref[...] = binc + jnp.broadcast_to(inv, (B, K)).astype(jnp.int32)


_select = pl.pallas_call(
    _select_body,
    grid=(B,),
    in_specs=[pl.BlockSpec((1, C, D), lambda b: (b, 0, 0))],
    out_specs=pl.BlockSpec((B, K), lambda b: (0, 0)),
    out_shape=jax.ShapeDtypeStruct((B, K), jnp.int32),
    scratch_shapes=[pltpu.VMEM((1, C), jnp.float32)],
)


def _sc_gather_body(x_hbm, src_hbm, out_hbm, src_v, buf0, buf1, sem0, sem1):
    wid = lax.axis_index("s") * 2 + lax.axis_index("c")
    base = wid * ROWS_PER_W
    pltpu.sync_copy(src_hbm.at[pl.ds(base, ROWS_PER_W)], src_v)
    bufs = (buf0, buf1)
    sems = (sem0, sem1)
    copies = []
    for k in range(NCHUNK):
        idx = src_v.at[pl.ds(k * CHUNK, CHUNK)]
        copies.append(pltpu.async_copy(x_hbm.at[idx], bufs[k % 2], sems[k % 2]))
        if k > 0:
            copies[k - 1].wait()
            pltpu.sync_copy(bufs[(k - 1) % 2],
                            out_hbm.at[pl.ds(base + (k - 1) * CHUNK, CHUNK)])
    copies[NCHUNK - 1].wait()
    pltpu.sync_copy(bufs[(NCHUNK - 1) % 2],
                    out_hbm.at[pl.ds(base + (NCHUNK - 1) * CHUNK, CHUNK)])


@functools.cache
def _sc_gather():
    mesh = plsc.VectorSubcoreMesh(core_axis_name="c", subcore_axis_name="s")
    return pl.kernel(
        _sc_gather_body,
        out_type=jax.ShapeDtypeStruct((B * K, D), jnp.float32),
        mesh=mesh,
        scratch_types=[
            pltpu.VMEM((ROWS_PER_W,), jnp.int32),
            pltpu.VMEM((CHUNK, D), jnp.float32),
            pltpu.VMEM((CHUNK, D), jnp.float32),
            pltpu.SemaphoreType.DMA,
            pltpu.SemaphoreType.DMA,
        ],
    )


def kernel(x):
    x3 = x.reshape(B, C, D)
    src = _select(x3)
    out = _sc_gather()(x3.reshape(B * C, D), src.reshape(B * K))
    return out.reshape(B, K, H, W)


# speedup vs baseline: 1.1606x; 1.1606x over previous
import functools

import jax
import jax.numpy as jnp
from jax import lax
from jax.experimental import pallas as pl
from jax.experimental.pallas import tpu as pltpu
from jax.experimental.pallas import tpu_sc as plsc

B, C, H, W = 16, 768, 48, 48
K = 192
NW = 32
NCT = C // NW
GRP = 8
NGRP = NCT // GRP
NCHK = B * NGRP
NJ = K // NW


def _sc_energy_body(x_hbm, e_hbm, buf0, buf1, resbuf, sem0, sem1):
    wid = lax.axis_index("s") * 2 + lax.axis_index("c")
    c0 = wid * NCT
    bufs = (buf0, buf1)
    sems = (sem0, sem1)
    for nb in range(2):
        pltpu.async_copy(x_hbm.at[nb // NGRP, pl.ds(c0 + (nb % NGRP) * GRP, GRP)],
                         bufs[nb], sems[nb])
    zero = jnp.zeros((16,), jnp.float32)

    def clr(r, carry):
        resbuf[r] = zero
        return carry

    lax.fori_loop(0, NCT, clr, 0)

    def outer(go, carry):
        for nb in range(2):
            s = go * 2 + nb
            b = s // NGRP
            g = s - b * NGRP
            pltpu.make_async_copy(x_hbm.at[0, pl.ds(0, GRP)],
                                  bufs[nb], sems[nb]).wait()
            for p in range(GRP):
                def inner(hq, acc, _p=p, _nb=nb):
                    a0, a1, a2 = acc
                    for hh in range(8):
                        h = hq * 8 + hh
                        r0 = bufs[_nb][_p, h, pl.ds(0, 16)]
                        r1 = bufs[_nb][_p, h, pl.ds(16, 16)]
                        r2 = bufs[_nb][_p, h, pl.ds(32, 16)]
                        a0 = a0 + r0 * r0
                        a1 = a1 + r1 * r1
                        a2 = a2 + r2 * r2
                    return (a0, a1, a2)

                a0, a1, a2 = lax.fori_loop(0, H // 8, inner, (zero, zero, zero))
                ch = g * GRP + p
                resbuf[ch] = resbuf[ch] + (a0 + a1 + a2)

            @pl.when(s + 2 < NCHK)
            def _(s=s, nb=nb):
                s2 = s + 2
                b2 = s2 // NGRP
                g2 = s2 - b2 * NGRP
                pltpu.async_copy(x_hbm.at[b2, pl.ds(c0 + g2 * GRP, GRP)],
                                 bufs[nb], sems[nb])
        return carry

    lax.fori_loop(0, NCHK // 2, outer, 0)
    pltpu.sync_copy(resbuf, e_hbm.at[pl.ds(c0, NCT)])


@functools.cache
def _sc_energy():
    mesh = plsc.VectorSubcoreMesh(core_axis_name="c", subcore_axis_name="s")
    return pl.kernel(
        _sc_energy_body,
        out_type=jax.ShapeDtypeStruct((C, 16), jnp.float32),
        mesh=mesh,
        scratch_types=[
            pltpu.VMEM((GRP, H, W), jnp.float32),
            pltpu.VMEM((GRP, H, W), jnp.float32),
            pltpu.VMEM((NCT, 16), jnp.float32),
            pltpu.SemaphoreType.DMA,
            pltpu.SemaphoreType.DMA,
        ],
    )


def _select_body(e_ref, src_ref):
    ecol = jnp.sum(e_ref[...], axis=1, keepdims=True)
    row_i = lax.broadcasted_iota(jnp.int32, (C, C), 0)
    col_i = lax.broadcasted_iota(jnp.int32, (C, C), 1)
    eye = (row_i == col_i).astype(jnp.float32)

    def split3(v):
        hi = v.astype(jnp.bfloat16).astype(jnp.float32)
        r1 = v - hi
        mid = r1.astype(jnp.bfloat16).astype(jnp.float32)
        return hi, mid, r1 - mid

    def transpose_row(v):
        out = jnp.zeros((C, 1), jnp.float32)
        for p in split3(v):
            out = out + lax.dot_general(eye, p, (((1,), (1,)), ((), ())),
                                        preferred_element_type=jnp.float32)
        return out

    def transpose_col(v):
        out = jnp.zeros((1, C), jnp.float32)
        for p in split3(v):
            out = out + lax.dot_general(p, eye, (((0,), (0,)), ((), ())),
                                        preferred_element_type=jnp.float32)
        return out

    erow = transpose_col(ecol)
    gt = ecol > erow
    eq = ecol == erow
    jlt = row_i < col_i
    m = jnp.where(gt | (eq & jlt), 1.0, 0.0)
    ones = jnp.ones((1, C), jnp.float32)
    rank = lax.dot_general(ones, m, (((1,), (0,)), ((), ())),
                           preferred_element_type=jnp.float32)
    rank_col = transpose_row(rank)
    slot = lax.broadcasted_iota(jnp.int32, (C, K), 1).astype(jnp.float32)
    onehot = jnp.where(rank_col == slot, 1.0, 0.0)
    cids = lax.broadcasted_iota(jnp.int32, (1, C), 1).astype(jnp.float32)
    inv = jnp.zeros((1, K), jnp.float32)
    for p in split3(cids):
        inv = inv + lax.dot_general(p, onehot, (((1,), (0,)), ((), ())),
                                    preferred_element_type=jnp.float32)
    src_ref[...] = inv.astype(jnp.int32)


_select = pl.pallas_call(
    _select_body,
    out_shape=jax.ShapeDtypeStruct((1, K), jnp.int32),
)


def _sc_gather_body(x_hbm, inv_hbm, out_hbm, inv_v, buf0, buf1, sem0, sem1):
    wid = lax.axis_index("s") * 2 + lax.axis_index("c")
    pltpu.sync_copy(inv_hbm, inv_v.at[pl.ds(0, K)])
    cvec = inv_v[pl.ds(NJ * wid, 16)]
    bufs = (buf0, buf1)
    sems = (sem0, sem1)
    copies = []
    NS = 2 * NJ
    for s in range(NS):
        t, hb = divmod(s, 2)
        c = cvec[t]
        copies.append(pltpu.async_copy(
            x_hbm.at[pl.ds(hb * (B // 2), B // 2), c], bufs[s % 2], sems[s % 2]))
        if s > 0:
            tp, hp = divmod(s - 1, 2)
            copies[s - 1].wait()
            pltpu.sync_copy(bufs[(s - 1) % 2],
                            out_hbm.at[pl.ds(hp * (B // 2), B // 2),
                                       NJ * wid + tp])
    tp, hp = divmod(NS - 1, 2)
    copies[NS - 1].wait()
    pltpu.sync_copy(bufs[(NS - 1) % 2],
                    out_hbm.at[pl.ds(hp * (B // 2), B // 2), NJ * wid + tp])


@functools.cache
def _sc_gather():
    mesh = plsc.VectorSubcoreMesh(core_axis_name="c", subcore_axis_name="s")
    return pl.kernel(
        _sc_gather_body,
        out_type=jax.ShapeDtypeStruct((B, K, H, W), jnp.float32),
        mesh=mesh,
        scratch_types=[
            pltpu.VMEM((K + 16,), jnp.int32),
            pltpu.VMEM((B // 2, H, W), jnp.float32),
            pltpu.VMEM((B // 2, H, W), jnp.float32),
            pltpu.SemaphoreType.DMA,
            pltpu.SemaphoreType.DMA,
        ],
    )


def kernel(x):
    e2 = _sc_energy()(x)
    inv = _select(e2)
    return _sc_gather()(x, inv.reshape(K))


# speedup vs baseline: 1.2069x; 1.0399x over previous
import functools

import jax
import jax.numpy as jnp
from jax import lax
from jax.experimental import pallas as pl
from jax.experimental.pallas import tpu as pltpu
from jax.experimental.pallas import tpu_sc as plsc

B, C, H, W = 16, 768, 48, 48
K = 192
NW = 32
CT = 256
NCT = (C - CT) // NW
GRP = 8
NGRP = NCT // GRP
NCHK = B * NGRP
NJ = K // NW


def _sc_energy_body(x_hbm, e_hbm, buf0, buf1, resbuf, sem0, sem1):
    wid = lax.axis_index("s") * 2 + lax.axis_index("c")
    c0 = CT + wid * NCT
    bufs = (buf0, buf1)
    sems = (sem0, sem1)
    for nb in range(2):
        pltpu.async_copy(x_hbm.at[nb // NGRP, pl.ds(c0 + (nb % NGRP) * GRP, GRP)],
                         bufs[nb], sems[nb])
    zero = jnp.zeros((16,), jnp.float32)

    def clr(r, carry):
        resbuf[r] = zero
        return carry

    lax.fori_loop(0, NCT, clr, 0)

    def outer(go, carry):
        for nb in range(2):
            s = go * 2 + nb
            b = s // NGRP
            g = s - b * NGRP
            pltpu.make_async_copy(x_hbm.at[0, pl.ds(0, GRP)],
                                  bufs[nb], sems[nb]).wait()
            for p in range(GRP):
                def inner(hq, acc, _p=p, _nb=nb):
                    a0, a1, a2 = acc
                    for hh in range(8):
                        h = hq * 8 + hh
                        r0 = bufs[_nb][_p, h, pl.ds(0, 16)]
                        r1 = bufs[_nb][_p, h, pl.ds(16, 16)]
                        r2 = bufs[_nb][_p, h, pl.ds(32, 16)]
                        a0 = a0 + r0 * r0
                        a1 = a1 + r1 * r1
                        a2 = a2 + r2 * r2
                    return (a0, a1, a2)

                a0, a1, a2 = lax.fori_loop(0, H // 8, inner, (zero, zero, zero))
                ch = g * GRP + p
                resbuf[ch] = resbuf[ch] + (a0 + a1 + a2)

            @pl.when(s + 2 < NCHK)
            def _(s=s, nb=nb):
                s2 = s + 2
                b2 = s2 // NGRP
                g2 = s2 - b2 * NGRP
                pltpu.async_copy(x_hbm.at[b2, pl.ds(c0 + g2 * GRP, GRP)],
                                 bufs[nb], sems[nb])
        return carry

    lax.fori_loop(0, NCHK // 2, outer, 0)
    pltpu.sync_copy(resbuf, e_hbm.at[pl.ds(wid * NCT, NCT)])


@functools.cache
def _sc_energy():
    mesh = plsc.VectorSubcoreMesh(core_axis_name="c", subcore_axis_name="s")
    return pl.kernel(
        _sc_energy_body,
        out_type=jax.ShapeDtypeStruct((C - CT, 16), jnp.float32),
        mesh=mesh,
        scratch_types=[
            pltpu.VMEM((GRP, H, W), jnp.float32),
            pltpu.VMEM((GRP, H, W), jnp.float32),
            pltpu.VMEM((NCT, 16), jnp.float32),
            pltpu.SemaphoreType.DMA,
            pltpu.SemaphoreType.DMA,
        ],
    )


def _tc_energy_body(x_ref, e_ref, eacc):
    b = pl.program_id(0)
    xb = x_ref[...]
    s = jnp.sum(xb * xb, axis=(0, 2, 3))[None, :]

    @pl.when(b == 0)
    def _():
        eacc[...] = s

    @pl.when(b > 0)
    def _():
        eacc[...] = eacc[...] + s

    @pl.when(b == B - 1)
    def _():
        e_ref[...] = eacc[...]


_tc_energy = pl.pallas_call(
    _tc_energy_body,
    grid=(B,),
    in_specs=[pl.BlockSpec((1, CT, H, W), lambda b: (b, 0, 0, 0))],
    out_specs=pl.BlockSpec((1, CT), lambda b: (0, 0)),
    out_shape=jax.ShapeDtypeStruct((1, CT), jnp.float32),
    scratch_shapes=[pltpu.VMEM((1, CT), jnp.float32)],
)


def _select_body(etc_ref, esc_ref, src_ref):
    ecol_sc = jnp.sum(esc_ref[...], axis=1, keepdims=True)
    row_i = lax.broadcasted_iota(jnp.int32, (C, C), 0)
    col_i = lax.broadcasted_iota(jnp.int32, (C, C), 1)
    eye = (row_i == col_i).astype(jnp.float32)
    eye_t = (lax.broadcasted_iota(jnp.int32, (CT, CT), 0) ==
             lax.broadcasted_iota(jnp.int32, (CT, CT), 1)).astype(jnp.float32)

    def split3(v):
        hi = v.astype(jnp.bfloat16).astype(jnp.float32)
        r1 = v - hi
        mid = r1.astype(jnp.bfloat16).astype(jnp.float32)
        return hi, mid, r1 - mid

    def transpose_row(v, n, ey):
        out = jnp.zeros((n, 1), jnp.float32)
        for p in split3(v):
            out = out + lax.dot_general(ey, p, (((1,), (1,)), ((), ())),
                                        preferred_element_type=jnp.float32)
        return out

    ecol = jnp.concatenate(
        [transpose_row(etc_ref[...], CT, eye_t), ecol_sc], axis=0)

    def transpose_col(v):
        out = jnp.zeros((1, C), jnp.float32)
        for p in split3(v):
            out = out + lax.dot_general(p, eye, (((0,), (0,)), ((), ())),
                                        preferred_element_type=jnp.float32)
        return out

    erow = transpose_col(ecol)
    gt = ecol > erow
    eq = ecol == erow
    jlt = row_i < col_i
    m = jnp.where(gt | (eq & jlt), 1.0, 0.0)
    ones = jnp.ones((1, C), jnp.float32)
    rank = lax.dot_general(ones, m, (((1,), (0,)), ((), ())),
                           preferred_element_type=jnp.float32)
    rank_col = transpose_row(rank, C, eye)
    slot = lax.broadcasted_iota(jnp.int32, (C, K), 1).astype(jnp.float32)
    onehot = jnp.where(rank_col == slot, 1.0, 0.0)
    cids = lax.broadcasted_iota(jnp.int32, (1, C), 1).astype(jnp.float32)
    inv = jnp.zeros((1, K), jnp.float32)
    for p in split3(cids):
        inv = inv + lax.dot_general(p, onehot, (((1,), (0,)), ((), ())),
                                    preferred_element_type=jnp.float32)
    src_ref[...] = inv.astype(jnp.int32)


_select = pl.pallas_call(
    _select_body,
    out_shape=jax.ShapeDtypeStruct((1, K), jnp.int32),
)


def _sc_gather_body(x_hbm, inv_hbm, out_hbm, inv_v, buf0, buf1, sem0, sem1):
    wid = lax.axis_index("s") * 2 + lax.axis_index("c")
    pltpu.sync_copy(inv_hbm, inv_v.at[pl.ds(0, K)])
    cvec = inv_v[pl.ds(NJ * wid, 16)]
    bufs = (buf0, buf1)
    sems = (sem0, sem1)
    copies = []
    NS = 2 * NJ
    for s in range(NS):
        t, hb = divmod(s, 2)
        c = cvec[t]
        copies.append(pltpu.async_copy(
            x_hbm.at[pl.ds(hb * (B // 2), B // 2), c], bufs[s % 2], sems[s % 2]))
        if s > 0:
            tp, hp = divmod(s - 1, 2)
            copies[s - 1].wait()
            pltpu.sync_copy(bufs[(s - 1) % 2],
                            out_hbm.at[pl.ds(hp * (B // 2), B // 2),
                                       NJ * wid + tp])
    tp, hp = divmod(NS - 1, 2)
    copies[NS - 1].wait()
    pltpu.sync_copy(bufs[(NS - 1) % 2],
                    out_hbm.at[pl.ds(hp * (B // 2), B // 2), NJ * wid + tp])


@functools.cache
def _sc_gather():
    mesh = plsc.VectorSubcoreMesh(core_axis_name="c", subcore_axis_name="s")
    return pl.kernel(
        _sc_gather_body,
        out_type=jax.ShapeDtypeStruct((B, K, H, W), jnp.float32),
        mesh=mesh,
        scratch_types=[
            pltpu.VMEM((K + 16,), jnp.int32),
            pltpu.VMEM((B // 2, H, W), jnp.float32),
            pltpu.VMEM((B // 2, H, W), jnp.float32),
            pltpu.SemaphoreType.DMA,
            pltpu.SemaphoreType.DMA,
        ],
    )


def kernel(x):
    e_sc = _sc_energy()(x)
    e_tc = _tc_energy(x)
    inv = _select(e_tc, e_sc)
    return _sc_gather()(x, inv.reshape(K))
